# baseline (speedup 1.0000x reference)
import functools

import jax
import jax.numpy as jnp
from jax import lax
from jax.experimental import pallas as pl
from jax.experimental.pallas import tpu as pltpu
from jax.experimental.pallas import tpu_sc as plsc

N = 9999
E = 319968
D_IN = 128
H = 256
H2 = 128
NPART = 3
BGRAPH = N // NPART

N_PAD = 10240
E_PAD = 327680
K = 128
R = 512
NB = N_PAD // R
ROWS_T = N_PAD // 16
DW = 16

_MESH = dict(core_axis_name="c", subcore_axis_name="s", num_cores=2,
             num_subcores=16)


def _deg_kernel(dst_rows):
  ng = 16
  nch = (E_PAD // 2) // 16 // K
  ngrp = nch // ng
  rows_half = (E_PAD // K) // 2

  @functools.partial(
      pl.kernel,
      mesh=plsc.VectorSubcoreMesh(**_MESH),
      out_type=jax.ShapeDtypeStruct((2 * N_PAD, DW), jnp.float32),
      scratch_types=[
          pltpu.VMEM((ng, K), jnp.int32),
          pltpu.VMEM((K, DW), jnp.float32),
          pltpu.VMEM_SHARED((N_PAD, DW), jnp.float32),
          pltpu.SemaphoreType.DMA,
          pltpu.SemaphoreType.DMA,
      ],
  )
  def deg(dst_hbm, out_hbm, idx_v, ones_v, acc, isem, ssem):
    c = lax.axis_index("c")
    t = lax.axis_index("s")

    @pl.loop(0, K)
    def _(i):
      @pl.loop(0, DW // 16)
      def _(j):
        ones_v[i, pl.ds(j * 16, 16)] = jnp.full((16,), 1.0, jnp.float32)

    @pl.loop(0, ROWS_T // K)
    def _(i):
      pltpu.sync_copy(ones_v, acc.at[pl.ds(t * ROWS_T + i * K, K)])

    plsc.subcore_barrier()

    @pl.loop(0, ngrp)
    def _(g):
      @pl.when(g > 0)
      def _():
        for j in range(ng):
          pltpu.make_async_copy(ones_v, acc.at[idx_v.at[j]], ssem).wait()

      row0 = c * rows_half + t * ngrp * ng + g * ng
      pltpu.async_copy(dst_hbm.at[pl.ds(row0, ng)], idx_v, isem).wait()
      for j in range(ng):
        pltpu.async_copy(ones_v, acc.at[idx_v.at[j]], ssem, add=True)

    for j in range(ng):
      pltpu.make_async_copy(ones_v, acc.at[idx_v.at[j]], ssem).wait()

    plsc.subcore_barrier()
    pltpu.sync_copy(acc.at[pl.ds(t * ROWS_T, ROWS_T)],
                    out_hbm.at[pl.ds(c * N_PAD + t * ROWS_T, ROWS_T)])

  return deg(dst_rows)


def _make_scatter(hc, edge_split, ng=16, nbuf=2):
  nch = E_PAD // (2 if edge_split else 1) // 16 // K
  ngrp = nch // ng
  assert ngrp * ng == nch and ng % nbuf == 0

  if edge_split:
    out_t = [jax.ShapeDtypeStruct((N_PAD, hc), jnp.float32),
             jax.ShapeDtypeStruct((N_PAD, hc), jnp.float32)]
  else:
    out_t = jax.ShapeDtypeStruct((2 * N_PAD, hc), jnp.float32)

  @functools.partial(
      pl.kernel,
      mesh=plsc.VectorSubcoreMesh(**_MESH),
      out_type=out_t,
      scratch_types=[
          pltpu.VMEM((ng, K), jnp.int32),
          pltpu.VMEM((ng, K), jnp.int32),
      ] + [pltpu.VMEM((K, hc), jnp.float32) for _ in range(nbuf)] + [
          pltpu.VMEM_SHARED((N_PAD, hc), jnp.float32),
          pltpu.SemaphoreType.DMA,
      ] + [pltpu.SemaphoreType.DMA for _ in range(2 * nbuf)],
  )
  def scat(g_hbm, srcr_hbm, dstr_hbm, *args):
    if edge_split:
      outs = args[:2]
      rest = args[2:]
    else:
      outs = args[:1]
      rest = args[1:]
    src_v, dst_v = rest[0], rest[1]
    rest = rest[2:]
    rows = rest[:nbuf]
    acc = rest[nbuf]
    isem = rest[nbuf + 1]
    gsems = rest[nbuf + 2:nbuf + 2 + nbuf]
    ssems = rest[nbuf + 2 + nbuf:]
    c = lax.axis_index("c")
    t = lax.axis_index("s")

    def drain_scatter(b, j):
      pltpu.make_async_copy(rows[b], acc.at[dst_v.at[j]], ssems[b]).wait()

    if edge_split:
      @pl.loop(0, K)
      def _(i):
        @pl.loop(0, hc // 16)
        def _(j):
          rows[0][i, pl.ds(j * 16, 16)] = jnp.zeros((16,), jnp.float32)

      @pl.loop(0, ROWS_T // K)
      def _(i):
        pltpu.sync_copy(rows[0], acc.at[pl.ds(t * ROWS_T + i * K, K)])
      out_base = 0
      row_base = c * (E_PAD // K // 2) + t * nch
      srow_base = row_base
    else:
      pltpu.sync_copy(g_hbm.at[pl.ds(c * N_PAD + t * ROWS_T, ROWS_T)],
                      acc.at[pl.ds(t * ROWS_T, ROWS_T)])
      out_base = c * N_PAD
      row_base = t * nch
      srow_base = c * (E_PAD // K) + row_base

    plsc.subcore_barrier()

    @pl.loop(0, ngrp)
    def _(g):
      @pl.when(g > 0)
      def _():
        for b in range(nbuf):
          drain_scatter(b, ng - nbuf + b)

      d1 = pltpu.async_copy(srcr_hbm.at[pl.ds(srow_base + g * ng, ng)],
                            src_v, isem)
      d2 = pltpu.async_copy(dstr_hbm.at[pl.ds(row_base + g * ng, ng)],
                            dst_v, isem)
      d1.wait()
      d2.wait()
      prev = None
      for j in range(ng):
        b = j % nbuf
        if j >= nbuf:
          drain_scatter(b, j - nbuf)
        gd = pltpu.async_copy(g_hbm.at[src_v.at[j]], rows[b], gsems[b])
        if prev is not None:
          prev[0].wait()
          pltpu.async_copy(rows[prev[1]], acc.at[dst_v.at[prev[2]]],
                           ssems[prev[1]], add=True)
        prev = (gd, b, j)
      prev[0].wait()
      pltpu.async_copy(rows[prev[1]], acc.at[dst_v.at[prev[2]]],
                       ssems[prev[1]], add=True)

    for b in range(nbuf):
      drain_scatter(b, ng - nbuf + b)

    plsc.subcore_barrier()
    if edge_split:
      @pl.when(c == 0)
      def _():
        pltpu.sync_copy(acc.at[pl.ds(t * ROWS_T, ROWS_T)],
                        outs[0].at[pl.ds(t * ROWS_T, ROWS_T)])

      @pl.when(c == 1)
      def _():
        pltpu.sync_copy(acc.at[pl.ds(t * ROWS_T, ROWS_T)],
                        outs[1].at[pl.ds(t * ROWS_T, ROWS_T)])
    else:
      pltpu.sync_copy(acc.at[pl.ds(t * ROWS_T, ROWS_T)],
                      outs[0].at[pl.ds(out_base + t * ROWS_T, ROWS_T)])

  return scat


_scatter_h = _make_scatter(H // 2, edge_split=False)
_scatter_h2 = _make_scatter(H2, edge_split=True)


def _layer1(x_pad, w1, degp):
  hc = H // 2

  def body(x_ref, w_ref, da_ref, db_ref, g_ref, dinv_ref):
    d = da_ref[:, 0:1] + db_ref[:, 0:1] - 1.0
    dv = lax.rsqrt(d)
    dinv_ref[...] = dv
    g_ref[...] = jnp.dot(x_ref[...], w_ref[...],
                         preferred_element_type=jnp.float32) * dv

  return pl.pallas_call(
      body,
      grid=(NB, 2),
      in_specs=[
          pl.BlockSpec((R, D_IN), lambda i, j: (i, 0)),
          pl.BlockSpec((D_IN, hc), lambda i, j: (0, j)),
          pl.BlockSpec((R, DW), lambda i, j: (i, 0)),
          pl.BlockSpec((R, DW), lambda i, j: (i + NB, 0)),
      ],
      out_specs=[
          pl.BlockSpec((R, hc), lambda i, j: (j * NB + i, 0)),
          pl.BlockSpec((R, 1), lambda i, j: (i, 0)),
      ],
      out_shape=[
          jax.ShapeDtypeStruct((2 * N_PAD, hc), jnp.float32),
          jax.ShapeDtypeStruct((N_PAD, 1), jnp.float32),
      ],
  )(x_pad, w1, degp, degp)


def _layer_mid(spg, dinv, b, w, h_in, h_out):
  hci = h_in // 2
  hco = h_out // 2

  def body(sa_ref, sb_ref, dv_ref, b_ref, w_ref, g_ref):
    dv = dv_ref[...]
    p0 = jnp.maximum(sa_ref[...] * dv + b_ref[:, :hci], 0.0)
    p1 = jnp.maximum(sb_ref[...] * dv + b_ref[:, hci:], 0.0)
    acc = (jnp.dot(p0, w_ref[0, :hci, :],
                   preferred_element_type=jnp.float32) +
           jnp.dot(p1, w_ref[0, hci:, :],
                   preferred_element_type=jnp.float32))
    g_ref[...] = acc * dv

  w_stack = jnp.stack([w[:, :hco], w[:, hco:]])
  return pl.pallas_call(
      body,
      grid=(NB, 2),
      in_specs=[
          pl.BlockSpec((R, hci), lambda i, j: (i, 0)),
          pl.BlockSpec((R, hci), lambda i, j: (i + NB, 0)),
          pl.BlockSpec((R, 1), lambda i, j: (i, 0)),
          pl.BlockSpec((1, h_in), lambda i, j: (0, 0)),
          pl.BlockSpec((1, h_in, hco), lambda i, j: (j, 0, 0)),
      ],
      out_specs=pl.BlockSpec((R, hco), lambda i, j: (j * NB + i, 0)),
      out_shape=jax.ShapeDtypeStruct((2 * N_PAD, hco), jnp.float32),
  )(spg, spg, dinv, b.reshape(1, h_in), w_stack)


def _layer3(spg2, dinv, b2, w3):
  hci = H // 2

  def body(sa_ref, sb_ref, dv_ref, b_ref, w_ref, g_ref):
    dv = dv_ref[...]
    p0 = jnp.maximum(sa_ref[...] * dv + b_ref[:, :hci], 0.0)
    p1 = jnp.maximum(sb_ref[...] * dv + b_ref[:, hci:], 0.0)
    acc = (jnp.dot(p0, w_ref[:hci, :],
                   preferred_element_type=jnp.float32) +
           jnp.dot(p1, w_ref[hci:, :],
                   preferred_element_type=jnp.float32))
    g_ref[...] = acc * dv

  return pl.pallas_call(
      body,
      grid=(NB,),
      in_specs=[
          pl.BlockSpec((R, hci), lambda i: (i, 0)),
          pl.BlockSpec((R, hci), lambda i: (i + NB, 0)),
          pl.BlockSpec((R, 1), lambda i: (i, 0)),
          pl.BlockSpec((1, H), lambda i: (0, 0)),
          pl.BlockSpec((H, H2), lambda i: (0, 0)),
      ],
      out_specs=pl.BlockSpec((R, H2), lambda i: (i, 0)),
      out_shape=jax.ShapeDtypeStruct((N_PAD, H2), jnp.float32),
  )(spg2, spg2, dinv, b2.reshape(1, H), w3)


def _final_classifier(spg3a, spg3b, g3, dinv, b3,
                      wc1, bc1, wc2, bc2, wc3, bc3):
  din = H2 * NPART
  rg = 256
  rn = NPART * rg

  def body(sa_ref, sb_ref, g_ref, dv_ref, b_ref, w1_ref, b1_ref, w2_ref,
           b2_ref, w3_ref, b3c_ref, o_ref):
    s = sa_ref[...] + sb_ref[...] + g_ref[...]
    h = jnp.maximum(s * dv_ref[...] + b_ref[...], 0.0)
    hr = h.reshape(rg, din)
    z = jnp.maximum(
        jnp.dot(hr, w1_ref[...], preferred_element_type=jnp.float32)
        + b1_ref[...], 0.0)
    z = jnp.maximum(
        jnp.dot(z, w2_ref[...], preferred_element_type=jnp.float32)
        + b2_ref[...], 0.0)
    lg = (jnp.dot(z, w3_ref[...], preferred_element_type=jnp.float32)
          + b3c_ref[...])
    m = jnp.max(lg, axis=1, keepdims=True)
    lse = m + jnp.log(jnp.sum(jnp.exp(lg - m), axis=1, keepdims=True))
    o_ref[...] = lg - lse

  g = pl.cdiv(BGRAPH, rg)
  return pl.pallas_call(
      body,
      grid=(g,),
      in_specs=[
          pl.BlockSpec((rn, H2), lambda i: (i, 0)),
          pl.BlockSpec((rn, H2), lambda i: (i, 0)),
          pl.BlockSpec((rn, H2), lambda i: (i, 0)),
          pl.BlockSpec((rn, 1), lambda i: (i, 0)),
          pl.BlockSpec((1, H2), lambda i: (0, 0)),
          pl.BlockSpec((din, H), lambda i: (0, 0)),
          pl.BlockSpec((1, H), lambda i: (0, 0)),
          pl.BlockSpec((H, H2), lambda i: (0, 0)),
          pl.BlockSpec((1, H2), lambda i: (0, 0)),
          pl.BlockSpec((H2, 2), lambda i: (0, 0)),
          pl.BlockSpec((1, 2), lambda i: (0, 0)),
      ],
      out_specs=pl.BlockSpec((rg, 2), lambda i: (i, 0)),
      out_shape=jax.ShapeDtypeStruct((BGRAPH, 2), jnp.float32),
  )(spg3a, spg3b, g3, dinv, b3.reshape(1, H2), wc1, bc1.reshape(1, H),
    wc2, bc2.reshape(1, H2), wc3, bc3.reshape(1, 2))


def kernel(x, edge_index, batch, W1, b1, W2, b2, W3, b3,
           Wc1, bc1, Wc2, bc2, Wc3, bc3):
  del batch
  src = edge_index[0].astype(jnp.int32)
  dst = edge_index[1].astype(jnp.int32)

  extra = E_PAD - E
  pad_ids = N + 1 + (jnp.arange(extra, dtype=jnp.int32) % (N_PAD - N - 2))
  src_pad = jnp.concatenate([src, pad_ids])
  dst_pad = jnp.concatenate([dst, pad_ids])
  src_rows = src_pad.reshape(E_PAD // K, K)
  dst_rows = dst_pad.reshape(E_PAD // K, K)
  src2_rows = jnp.concatenate([src_rows, src_rows + N_PAD])

  degp = _deg_kernel(dst_rows)
  g1, dinv = _layer1(x, W1, degp)
  spg1 = _scatter_h(g1, src2_rows, dst_rows)
  g2 = _layer_mid(spg1, dinv, b1, W2, H, H)
  spg2 = _scatter_h(g2, src2_rows, dst_rows)
  g3 = _layer3(spg2, dinv, b2, W3)
  spg3a, spg3b = _scatter_h2(g3, src_rows, dst_rows)
  return _final_classifier(spg3a, spg3b, g3, dinv, b3,
                           Wc1, bc1, Wc2, bc2, Wc3, bc3)
